# baseline (speedup 1.0000x reference)
import functools

import jax
import jax.numpy as jnp
from jax import lax
from jax.experimental import pallas as pl
from jax.experimental.pallas import tpu as pltpu
from jax.experimental.pallas import tpu_sc as plsc

N_USERS = 25000
N_ITEMS = 25000
HALF = 25000
HP = 25088
RPT = HP // 16
PAD_OFF = HP - HALF
DUMMY = HALF
EMB = 64
N_EDGES = 800000
EPT = 51200
NE_PAD = 16 * EPT
CHUNK = 1024
SUB = 128
NSUB = CHUNK // SUB
NCHUNKS = EPT // CHUNK
CROWS = 112
FBUF = CHUNK + SUB + 16
TRASH = CHUNK + SUB


def _layer_body(combine, ego_hbm, src_hbm, dst_hbm, w_hbm, zeros_hbm,
                out_hbm, acc, srcv, dstv, wv, fsrc, fw, fidx, rows, idx2,
                semA, semB, semSA, semSB, semE):
  c = lax.axis_index("c")
  s = lax.axis_index("s")
  base_node = c * HALF
  r0 = s * RPT

  pltpu.sync_copy(zeros_hbm.at[pl.ds(r0, RPT)], acc.at[pl.ds(r0, RPT)])
  plsc.subcore_barrier()

  gsems = (semA, semB)
  ssems = (semSA, semSB)

  def edge_fetch(i, p):
    ebase = s * EPT + i * CHUNK
    pltpu.async_copy(src_hbm.at[pl.ds(ebase, CHUNK)], srcv.at[p], semE)
    pltpu.async_copy(dst_hbm.at[pl.ds(ebase, CHUNK)], dstv.at[p], semE)
    pltpu.async_copy(w_hbm.at[pl.ds(ebase, CHUNK)], wv.at[p], semE)

  edge_fetch(0, 0)

  def chunk_body(i, carry):
    p = lax.rem(i, 2)
    for _ in range(3):
      pltpu.make_async_copy(src_hbm.at[pl.ds(0, CHUNK)], srcv.at[p],
                            semE).wait()

    @pl.when(i + 1 < NCHUNKS)
    def _():
      edge_fetch(i + 1, 1 - p)

    zero16i = jnp.zeros((16,), jnp.int32)
    zero16f = jnp.zeros((16,), jnp.float32)
    dummy16 = jnp.full((16,), DUMMY, jnp.int32)
    lane = lax.iota(jnp.int32, 16)
    dnums = lax.GatherDimensionNumbers(
        offset_dims=(), collapsed_slice_dims=(0,), start_index_map=(0,))

    def prefix16(x):
      for k in (1, 2, 4, 8):
        sh = lax.gather(x, jnp.maximum(lane - k, 0)[:, None], dnums,
                        slice_sizes=(1,),
                        mode=lax.GatherScatterMode.PROMISE_IN_BOUNDS)
        x = x + jnp.where(lane >= k, sh, 0)
      return x

    off = jnp.int32(0)
    for b in range(CHUNK // 16):
      o = b * 16
      sv = srcv[p, pl.ds(o, 16)]
      sv = sv + jnp.where(sv >= HALF, PAD_OFF, 0).astype(jnp.int32)
      dv = dstv[p, pl.ds(o, 16)] - base_node
      ok = (dv >= 0) & (dv < HALF)
      cs = prefix16(ok.astype(jnp.int32))
      pos = jnp.where(ok, off + cs - 1, TRASH)
      plsc.store_scatter(fsrc, [pos], sv)
      plsc.store_scatter(fw, [pos], wv[p, pl.ds(o, 16)])
      plsc.store_scatter(fidx, [pos], dv)
      off = off + cs[15]

    for q in range(SUB // 16):
      fsrc[pl.ds(off + q * 16, 16)] = zero16i
      fw[pl.ds(off + q * 16, 16)] = zero16f
      fidx[pl.ds(off + q * 16, 16)] = dummy16
    nsub = lax.div(off + (SUB - 1), SUB)

    def fire_gather(j):
      @pl.when(j < nsub)
      def _():
        pltpu.async_copy(ego_hbm.at[fsrc.at[pl.ds(j * SUB, SUB)]],
                         rows.at[j % 2], gsems[j % 2])

    def wait_gather(j):
      @pl.when(j < nsub)
      def _():
        pltpu.make_async_copy(ego_hbm.at[pl.ds(0, SUB)], rows.at[j % 2],
                              gsems[j % 2]).wait()

    def wait_scatter(j):
      if j < 0:
        return

      @pl.when(j < nsub)
      def _():
        pltpu.make_async_copy(ego_hbm.at[pl.ds(0, SUB)], rows.at[j % 2],
                              ssems[j % 2]).wait()

    fire_gather(0)
    for j in range(NSUB):
      jb = j % 2
      if j + 1 < NSUB:
        wait_scatter(j - 1)
        fire_gather(j + 1)
      wait_gather(j)

      @pl.when(j < nsub)
      def _(j=j, jb=jb):
        def mul_body(gi, mcarry):
          wg = fw[pl.ds(j * SUB + gi * 16, 16)]
          for l in range(16):
            e = gi * 16 + l
            w = wg[l]
            for k in range(4):
              rows[jb, e, pl.ds(k * 16, 16)] = (
                  rows[jb, e, pl.ds(k * 16, 16)] * w)
          return mcarry

        lax.fori_loop(0, SUB // 16, mul_body, 0)
        for q in range(SUB // 16):
          idx2[j, pl.ds(q * 16, 16)] = fidx[pl.ds(j * SUB + q * 16, 16)]
        pltpu.async_copy(rows.at[jb], acc.at[idx2.at[j]], ssems[jb],
                         add=True)

    wait_scatter(NSUB - 2)
    wait_scatter(NSUB - 1)
    return carry

  lax.fori_loop(0, NCHUNKS, chunk_body, 0)
  plsc.subcore_barrier()

  if not combine:
    pltpu.sync_copy(acc.at[pl.ds(r0, RPT)],
                    out_hbm.at[pl.ds(c * HP + r0, RPT)])
  else:
    for k in range(RPT // CROWS):
      r = r0 + k * CROWS
      pltpu.sync_copy(acc.at[pl.ds(r, CROWS)], rows.at[0, pl.ds(0, CROWS)])
      pltpu.sync_copy(ego_hbm.at[pl.ds(c * HP + r, CROWS)],
                      rows.at[1, pl.ds(0, CROWS)])

      def comb_body(e, ccarry):
        for kk in range(4):
          sl = pl.ds(kk * 16, 16)
          rows[0, e, sl] = (rows[0, e, sl] + rows[1, e, sl]) * 0.5
        return ccarry

      lax.fori_loop(0, CROWS, comb_body, 0, unroll=2)
      pltpu.sync_copy(rows.at[0, pl.ds(0, CROWS)],
                      out_hbm.at[pl.ds(c * HP + r, CROWS)])


def _make_layer(combine):
  mesh = plsc.VectorSubcoreMesh(core_axis_name="c", subcore_axis_name="s",
                                num_cores=2, num_subcores=16)
  return pl.kernel(
      functools.partial(_layer_body, combine),
      out_type=jax.ShapeDtypeStruct((2 * HP, EMB), jnp.float32),
      mesh=mesh,
      scratch_types=[
          pltpu.VMEM_SHARED((HP, EMB), jnp.float32),
          pltpu.VMEM((2, CHUNK), jnp.int32),
          pltpu.VMEM((2, CHUNK), jnp.int32),
          pltpu.VMEM((2, CHUNK), jnp.float32),
          pltpu.VMEM((FBUF,), jnp.int32),
          pltpu.VMEM((FBUF,), jnp.float32),
          pltpu.VMEM((FBUF,), jnp.int32),
          pltpu.VMEM((2, SUB, EMB), jnp.float32),
          pltpu.VMEM((NSUB, SUB), jnp.int32),
          pltpu.SemaphoreType.DMA,
          pltpu.SemaphoreType.DMA,
          pltpu.SemaphoreType.DMA,
          pltpu.SemaphoreType.DMA,
          pltpu.SemaphoreType.DMA,
      ],
      compiler_params=pltpu.CompilerParams(use_tc_tiling_on_sc=False,
                                           needs_layout_passes=False),
      name="lgcl_layer2" if combine else "lgcl_layer1",
  )


_layer1 = _make_layer(combine=False)
_layer2 = _make_layer(combine=True)


@jax.jit
def _lgcl(user_emb, item_emb, edge_index, edge_weight):
  src = edge_index[0].astype(jnp.int32)
  dst = edge_index[1].astype(jnp.int32)
  w = edge_weight.astype(jnp.float32)
  npad = NE_PAD - N_EDGES
  src = jnp.pad(src, (0, npad))
  dst = jnp.pad(dst, (0, npad))
  w = jnp.pad(w, (0, npad))
  ego = jnp.zeros((2 * HP, EMB), jnp.float32)
  ego = ego.at[0:HALF].set(user_emb).at[HP:HP + HALF].set(item_emb)
  zeros = jnp.zeros((HP, EMB), jnp.float32)
  l1 = _layer1(ego, src, dst, w, zeros)
  out = _layer2(l1, src, dst, w, zeros)
  return out[0:HALF], out[HP:HP + HALF]


def kernel(user_emb, item_emb, edge_index, edge_weight, perturbed=False):
  return _lgcl(user_emb, item_emb, edge_index, edge_weight)


# speedup vs baseline: 1.9061x; 1.9061x over previous
import functools

import jax
import jax.numpy as jnp
from jax import lax
from jax.experimental import pallas as pl
from jax.experimental.pallas import tpu as pltpu
from jax.experimental.pallas import tpu_sc as plsc

N_USERS = 25000
N_ITEMS = 25000
HALF = 25000
HP = 25088
RPT = HP // 16
PAD_OFF = HP - HALF
DUMMY = HALF
EMB = 64
N_EDGES = 800000
CHUNK = 768
SUB = 128
NSUB = CHUNK // SUB
NBUF = 3
NCHUNKS = 67
EPT = NSUB * SUB * NCHUNKS
NE_PAD = 16 * EPT
CROWS = 112


def _layer_body(combine, ego_hbm, src_hbm, dst_hbm, w_hbm, zeros_hbm,
                out_hbm, acc, srcv, dstv, wv, rows, idx2,
                semG0, semG1, semG2, semS0, semS1, semS2, semE):
  c = lax.axis_index("c")
  s = lax.axis_index("s")
  base_node = c * HALF
  r0 = s * RPT

  pltpu.sync_copy(zeros_hbm.at[pl.ds(r0, RPT)], acc.at[pl.ds(r0, RPT)])
  plsc.subcore_barrier()

  gsems = (semG0, semG1, semG2)
  ssems = (semS0, semS1, semS2)

  def edge_fetch(i, p):
    ebase = s * EPT + i * CHUNK
    pltpu.async_copy(src_hbm.at[pl.ds(ebase, CHUNK)], srcv.at[p], semE)
    pltpu.async_copy(dst_hbm.at[pl.ds(ebase, CHUNK)], dstv.at[p], semE)
    pltpu.async_copy(w_hbm.at[pl.ds(ebase, CHUNK)], wv.at[p], semE)

  def drain_scatter(b):
    pltpu.make_async_copy(ego_hbm.at[pl.ds(0, SUB)], rows.at[b],
                          ssems[b]).wait()

  edge_fetch(0, 0)

  def chunk_body(i, carry):
    p = lax.rem(i, 2)
    for _ in range(3):
      pltpu.make_async_copy(src_hbm.at[pl.ds(0, CHUNK)], srcv.at[p],
                            semE).wait()

    @pl.when(i + 1 < NCHUNKS)
    def _():
      edge_fetch(i + 1, 1 - p)

    for j in range(NSUB):
      for q in range(SUB // 16):
        o = j * SUB + q * 16
        sv = srcv[p, pl.ds(o, 16)]
        sv = sv + jnp.where(sv >= HALF, PAD_OFF, 0).astype(jnp.int32)
        srcv[p, pl.ds(o, 16)] = sv
        dv = dstv[p, pl.ds(o, 16)] - base_node
        ok = (dv >= 0) & (dv < HALF)
        idx2[p, j, pl.ds(q * 16, 16)] = jnp.where(ok, dv, DUMMY)

    def fire_gather(j):
      pltpu.async_copy(ego_hbm.at[srcv.at[p, pl.ds(j * SUB, SUB)]],
                       rows.at[j % NBUF], gsems[j % NBUF])

    scat = [None] * NBUF
    for j in range(2):
      @pl.when(i > 0)
      def _(b=j % NBUF):
        drain_scatter(b)
      fire_gather(j)

    for j in range(NSUB):
      b = j % NBUF
      if j + 2 < NSUB:
        nb = (j + 2) % NBUF
        if scat[nb] is not None:
          scat[nb].wait()
        else:
          @pl.when(i > 0)
          def _(nb=nb):
            drain_scatter(nb)
        fire_gather(j + 2)
      pltpu.make_async_copy(ego_hbm.at[pl.ds(0, SUB)], rows.at[b],
                            gsems[b]).wait()

      def mul_body(gi, mcarry):
        wg = wv[p, pl.ds(j * SUB + gi * 16, 16)]
        for l in range(16):
          e = gi * 16 + l
          w = wg[l]
          for k in range(4):
            rows[b, e, pl.ds(k * 16, 16)] = (
                rows[b, e, pl.ds(k * 16, 16)] * w)
        return mcarry

      lax.fori_loop(0, SUB // 16, mul_body, 0)
      scat[b] = pltpu.async_copy(rows.at[b], acc.at[idx2.at[p, j]],
                                 ssems[b], add=True)
    return carry

  lax.fori_loop(0, NCHUNKS, chunk_body, 0)
  for b in range(NBUF):
    drain_scatter(b)
  plsc.subcore_barrier()

  if not combine:
    pltpu.sync_copy(acc.at[pl.ds(r0, RPT)],
                    out_hbm.at[pl.ds(c * HP + r0, RPT)])
  else:
    for k in range(RPT // CROWS):
      r = r0 + k * CROWS
      pltpu.sync_copy(acc.at[pl.ds(r, CROWS)], rows.at[0, pl.ds(0, CROWS)])
      pltpu.sync_copy(ego_hbm.at[pl.ds(c * HP + r, CROWS)],
                      rows.at[1, pl.ds(0, CROWS)])

      def comb_body(e, ccarry):
        for kk in range(4):
          sl = pl.ds(kk * 16, 16)
          rows[0, e, sl] = (rows[0, e, sl] + rows[1, e, sl]) * 0.5
        return ccarry

      lax.fori_loop(0, CROWS, comb_body, 0, unroll=2)
      pltpu.sync_copy(rows.at[0, pl.ds(0, CROWS)],
                      out_hbm.at[pl.ds(c * HP + r, CROWS)])


def _make_layer(combine):
  mesh = plsc.VectorSubcoreMesh(core_axis_name="c", subcore_axis_name="s",
                                num_cores=2, num_subcores=16)
  return pl.kernel(
      functools.partial(_layer_body, combine),
      out_type=jax.ShapeDtypeStruct((2 * HP, EMB), jnp.float32),
      mesh=mesh,
      scratch_types=[
          pltpu.VMEM_SHARED((HP, EMB), jnp.float32),
          pltpu.VMEM((2, CHUNK), jnp.int32),
          pltpu.VMEM((2, CHUNK), jnp.int32),
          pltpu.VMEM((2, CHUNK), jnp.float32),
          pltpu.VMEM((NBUF, SUB, EMB), jnp.float32),
          pltpu.VMEM((2, NSUB, SUB), jnp.int32),
          pltpu.SemaphoreType.DMA,
          pltpu.SemaphoreType.DMA,
          pltpu.SemaphoreType.DMA,
          pltpu.SemaphoreType.DMA,
          pltpu.SemaphoreType.DMA,
          pltpu.SemaphoreType.DMA,
          pltpu.SemaphoreType.DMA,
      ],
      compiler_params=pltpu.CompilerParams(use_tc_tiling_on_sc=False),
      name="lgcl_layer2" if combine else "lgcl_layer1",
  )


_layer1 = _make_layer(combine=False)
_layer2 = _make_layer(combine=True)


@jax.jit
def _lgcl(user_emb, item_emb, edge_index, edge_weight):
  src = edge_index[0].astype(jnp.int32)
  dst = edge_index[1].astype(jnp.int32)
  w = edge_weight.astype(jnp.float32)
  npad = NE_PAD - N_EDGES
  src = jnp.pad(src, (0, npad))
  dst = jnp.pad(dst, (0, npad))
  w = jnp.pad(w, (0, npad))
  ego = jnp.zeros((2 * HP, EMB), jnp.float32)
  ego = ego.at[0:HALF].set(user_emb).at[HP:HP + HALF].set(item_emb)
  zeros = jnp.zeros((HP, EMB), jnp.float32)
  l1 = _layer1(ego, src, dst, w, zeros)
  out = _layer2(l1, src, dst, w, zeros)
  return out[0:HALF], out[HP:HP + HALF]


def kernel(user_emb, item_emb, edge_index, edge_weight, perturbed=False):
  return _lgcl(user_emb, item_emb, edge_index, edge_weight)


# speedup vs baseline: 2.1069x; 1.1053x over previous
import functools

import jax
import jax.numpy as jnp
from jax import lax
from jax.experimental import pallas as pl
from jax.experimental.pallas import tpu as pltpu
from jax.experimental.pallas import tpu_sc as plsc

N_USERS = 25000
N_ITEMS = 25000
HALF = 25000
HP = 25088
RPT = HP // 16
PAD_OFF = HP - HALF
DUMMY = HALF
EMB = 64
N_EDGES = 800000
CHUNK = 1024
SUB = 128
NSUB = CHUNK // SUB
NBUF = 2
NCHUNKS = 50
EPT = NSUB * SUB * NCHUNKS
NE_PAD = 16 * EPT
CROWS = 112


def _layer_body(combine, ego_hbm, src_hbm, dst_hbm, w_hbm, zeros_hbm,
                out_hbm, acc, srcv, dstv, wv, rows, idx2,
                semG0, semG1, semS0, semS1, semE):
  c = lax.axis_index("c")
  s = lax.axis_index("s")
  base_node = c * HALF
  r0 = s * RPT

  pltpu.sync_copy(zeros_hbm.at[pl.ds(r0, RPT)], acc.at[pl.ds(r0, RPT)])
  plsc.subcore_barrier()

  gsems = (semG0, semG1)
  ssems = (semS0, semS1)

  def edge_fetch(i, p):
    ebase = s * EPT + i * CHUNK
    pltpu.async_copy(src_hbm.at[pl.ds(ebase, CHUNK)], srcv.at[p], semE)
    pltpu.async_copy(dst_hbm.at[pl.ds(ebase, CHUNK)], dstv.at[p], semE)
    pltpu.async_copy(w_hbm.at[pl.ds(ebase, CHUNK)], wv.at[p], semE)

  def drain_scatter(b):
    pltpu.make_async_copy(ego_hbm.at[pl.ds(0, SUB)], rows.at[b],
                          ssems[b]).wait()

  edge_fetch(0, 0)

  def chunk_body(i, carry):
    p = lax.rem(i, 2)
    for _ in range(3):
      pltpu.make_async_copy(src_hbm.at[pl.ds(0, CHUNK)], srcv.at[p],
                            semE).wait()

    @pl.when(i + 1 < NCHUNKS)
    def _():
      edge_fetch(i + 1, 1 - p)

    for j in range(NSUB):
      for q in range(SUB // 16):
        o = j * SUB + q * 16
        sv = srcv[p, pl.ds(o, 16)]
        sv = sv + jnp.where(sv >= HALF, PAD_OFF, 0).astype(jnp.int32)
        srcv[p, pl.ds(o, 16)] = sv
        dv = dstv[p, pl.ds(o, 16)] - base_node
        ok = (dv >= 0) & (dv < HALF)
        idx2[p, j, pl.ds(q * 16, 16)] = jnp.where(ok, dv, DUMMY)

    def fire_gather(j):
      return pltpu.async_copy(ego_hbm.at[srcv.at[p, pl.ds(j * SUB, SUB)]],
                              rows.at[j % NBUF], gsems[j % NBUF])

    scat = [None] * NBUF
    gath = [None] * NBUF

    @pl.when(i > 0)
    def _():
      drain_scatter(0)
    gath[0] = fire_gather(0)

    for j in range(NSUB):
      b = j % NBUF
      if j + 1 < NSUB:
        nb = (j + 1) % NBUF
        if scat[nb] is not None:
          scat[nb].wait()
        else:
          @pl.when(i > 0)
          def _(nb=nb):
            drain_scatter(nb)
        gath[nb] = fire_gather(j + 1)
      gath[b].wait()

      def mul_body(gi, mcarry):
        wg = wv[p, pl.ds(j * SUB + gi * 16, 16)]
        for l in range(16):
          e = gi * 16 + l
          w = wg[l]
          for k in range(4):
            rows[b, e, pl.ds(k * 16, 16)] = (
                rows[b, e, pl.ds(k * 16, 16)] * w)
        return mcarry

      lax.fori_loop(0, SUB // 16, mul_body, 0)
      scat[b] = pltpu.async_copy(rows.at[b], acc.at[idx2.at[p, j]],
                                 ssems[b], add=True)
    return carry

  lax.fori_loop(0, NCHUNKS, chunk_body, 0)
  drain_scatter(0)
  drain_scatter(1)
  plsc.subcore_barrier()

  if not combine:
    pltpu.sync_copy(acc.at[pl.ds(r0, RPT)],
                    out_hbm.at[pl.ds(c * HP + r0, RPT)])
  else:
    for k in range(RPT // CROWS):
      r = r0 + k * CROWS
      pltpu.sync_copy(acc.at[pl.ds(r, CROWS)], rows.at[0, pl.ds(0, CROWS)])
      pltpu.sync_copy(ego_hbm.at[pl.ds(c * HP + r, CROWS)],
                      rows.at[1, pl.ds(0, CROWS)])

      def comb_body(e, ccarry):
        for kk in range(4):
          sl = pl.ds(kk * 16, 16)
          rows[0, e, sl] = (rows[0, e, sl] + rows[1, e, sl]) * 0.5
        return ccarry

      lax.fori_loop(0, CROWS, comb_body, 0, unroll=2)
      pltpu.sync_copy(rows.at[0, pl.ds(0, CROWS)],
                      out_hbm.at[pl.ds(c * HP + r, CROWS)])


def _make_layer(combine):
  mesh = plsc.VectorSubcoreMesh(core_axis_name="c", subcore_axis_name="s",
                                num_cores=2, num_subcores=16)
  return pl.kernel(
      functools.partial(_layer_body, combine),
      out_type=jax.ShapeDtypeStruct((2 * HP, EMB), jnp.float32),
      mesh=mesh,
      scratch_types=[
          pltpu.VMEM_SHARED((HP, EMB), jnp.float32),
          pltpu.VMEM((2, CHUNK), jnp.int32),
          pltpu.VMEM((2, CHUNK), jnp.int32),
          pltpu.VMEM((2, CHUNK), jnp.float32),
          pltpu.VMEM((NBUF, SUB, EMB), jnp.float32),
          pltpu.VMEM((2, NSUB, SUB), jnp.int32),
          pltpu.SemaphoreType.DMA,
          pltpu.SemaphoreType.DMA,
          pltpu.SemaphoreType.DMA,
          pltpu.SemaphoreType.DMA,
          pltpu.SemaphoreType.DMA,
      ],
      compiler_params=pltpu.CompilerParams(use_tc_tiling_on_sc=False),
      name="lgcl_layer2" if combine else "lgcl_layer1",
  )


_layer1 = _make_layer(combine=False)
_layer2 = _make_layer(combine=True)


@jax.jit
def _lgcl(user_emb, item_emb, edge_index, edge_weight):
  src = edge_index[0].astype(jnp.int32)
  dst = edge_index[1].astype(jnp.int32)
  w = edge_weight.astype(jnp.float32)
  npad = NE_PAD - N_EDGES
  src = jnp.pad(src, (0, npad))
  dst = jnp.pad(dst, (0, npad))
  w = jnp.pad(w, (0, npad))
  ego = jnp.zeros((2 * HP, EMB), jnp.float32)
  ego = ego.at[0:HALF].set(user_emb).at[HP:HP + HALF].set(item_emb)
  zeros = jnp.zeros((HP, EMB), jnp.float32)
  l1 = _layer1(ego, src, dst, w, zeros)
  out = _layer2(l1, src, dst, w, zeros)
  return out[0:HALF], out[HP:HP + HALF]


def kernel(user_emb, item_emb, edge_index, edge_weight, perturbed=False):
  return _lgcl(user_emb, item_emb, edge_index, edge_weight)
